# baseline (speedup 1.0000x reference)
import functools

import jax
import jax.numpy as jnp
from jax import lax
from jax.experimental import pallas as pl
from jax.experimental.pallas import tpu as pltpu
from jax.experimental.pallas import tpu_sc as plsc

NUM_EXPERTS = 16
TOP_K = 2
BLOCK_T = 1024

_INFO = plsc.get_sparse_core_info()
_NC, _NS, _L = _INFO.num_cores, _INFO.num_subcores, _INFO.num_lanes
NW = _NC * _NS

_GDN = lax.GatherDimensionNumbers(
    offset_dims=(), collapsed_slice_dims=(0,), start_index_map=(0,))


def _vgather(v, idx):
    return lax.gather(v, idx[:, None], _GDN, (1,),
                      mode=lax.GatherScatterMode.PROMISE_IN_BOUNDS)


def _matmul_body(x_ref, w_ref, out_ref):
    w = w_ref[...]
    tpw = out_ref.shape[2]
    for j in range(BLOCK_T // tpw):
        xj = x_ref[pl.ds(j * tpw, tpw), :]
        out_ref[j] = lax.dot_general(
            w, xj, (((1,), (1,)), ((), ())),
            preferred_element_type=jnp.float32)


def _sc_gate_body(l_hbm, vals_hbm, idx_hbm, tile, vflat, iflat):
    tpw = tile.shape[1]
    wid = lax.axis_index("s") * _NC + lax.axis_index("c")
    base = wid * tpw
    pltpu.sync_copy(l_hbm.at[wid], tile)
    lane = lax.iota(jnp.int32, _L)
    even = (lane & 1) == 0
    half = lane >> 1
    for g in range(tpw // _L):
        sl = pl.ds(g * _L, _L)
        ls = [tile[e, sl] for e in range(NUM_EXPERTS)]
        m = ls[0]
        for e in range(1, NUM_EXPERTS):
            m = jnp.maximum(m, ls[e])
        s = jnp.exp(ls[0] - m)
        for e in range(1, NUM_EXPERTS):
            s = s + jnp.exp(ls[e] - m)
        idx1 = jnp.full((_L,), NUM_EXPERTS, jnp.int32)
        for e in range(NUM_EXPERTS):
            ev = jnp.full((_L,), e, jnp.int32)
            idx1 = jnp.minimum(idx1, jnp.where(ls[e] == m, ev, NUM_EXPERTS))
        m2 = jnp.full((_L,), -jnp.inf, jnp.float32)
        for e in range(NUM_EXPERTS):
            ev = jnp.full((_L,), e, jnp.int32)
            m2 = jnp.maximum(m2, jnp.where(ev == idx1, -jnp.inf, ls[e]))
        idx2 = jnp.full((_L,), NUM_EXPERTS, jnp.int32)
        for e in range(NUM_EXPERTS):
            ev = jnp.full((_L,), e, jnp.int32)
            hit = jnp.logical_and(ls[e] == m2, ev != idx1)
            idx2 = jnp.minimum(idx2, jnp.where(hit, ev, NUM_EXPERTS))
        val1 = 1.0 / s
        val2 = jnp.exp(m2 - m) / s
        for out, a, b in ((vflat, val1, val2), (iflat, idx1, idx2)):
            lo = jnp.where(even, _vgather(a, half), _vgather(b, half))
            hi = jnp.where(even, _vgather(a, half + (_L // 2)),
                           _vgather(b, half + (_L // 2)))
            out[pl.ds(2 * g * _L, _L)] = lo
            out[pl.ds(2 * g * _L + _L, _L)] = hi
    pltpu.sync_copy(vflat, vals_hbm.at[pl.ds(TOP_K * base, TOP_K * tpw)])
    pltpu.sync_copy(iflat, idx_hbm.at[pl.ds(TOP_K * base, TOP_K * tpw)])


@jax.jit
def kernel(hidden_states, weight):
    x = hidden_states.reshape(-1, hidden_states.shape[-1])
    t, d = x.shape
    tpw = t // NW
    logits = pl.pallas_call(
        _matmul_body,
        grid=(t // BLOCK_T,),
        in_specs=[
            pl.BlockSpec((BLOCK_T, d), lambda i: (i, 0)),
            pl.BlockSpec((NUM_EXPERTS, d), lambda i: (0, 0)),
        ],
        out_specs=pl.BlockSpec(
            (BLOCK_T // tpw, NUM_EXPERTS, tpw), lambda i: (i, 0, 0)),
        out_shape=jax.ShapeDtypeStruct((NW, NUM_EXPERTS, tpw), jnp.float32),
    )(x, weight)

    sc_gate = functools.partial(
        pl.kernel,
        mesh=plsc.VectorSubcoreMesh(core_axis_name="c", subcore_axis_name="s"),
        out_type=[
            jax.ShapeDtypeStruct((t * TOP_K,), jnp.float32),
            jax.ShapeDtypeStruct((t * TOP_K,), jnp.int32),
        ],
        scratch_types=[
            pltpu.VMEM((NUM_EXPERTS, tpw), jnp.float32),
            pltpu.VMEM((TOP_K * tpw,), jnp.float32),
            pltpu.VMEM((TOP_K * tpw,), jnp.int32),
        ],
    )(_sc_gate_body)
    vals_flat, idx_flat = sc_gate(logits)
    return vals_flat.reshape(t, TOP_K), idx_flat.reshape(t, TOP_K)


# speedup vs baseline: 1.3142x; 1.3142x over previous
import functools

import jax
import jax.numpy as jnp
from jax import lax
from jax.experimental import pallas as pl
from jax.experimental.pallas import tpu as pltpu
from jax.experimental.pallas import tpu_sc as plsc

NUM_EXPERTS = 16
TOP_K = 2
BLOCK_T = 1024

_INFO = plsc.get_sparse_core_info()
_NC, _NS, _L = _INFO.num_cores, _INFO.num_subcores, _INFO.num_lanes
NW = _NC * _NS


def _matmul_body(x_ref, w_ref, out_ref):
    w = w_ref[...]
    tpw = out_ref.shape[2]
    for j in range(BLOCK_T // tpw):
        xj = x_ref[pl.ds(j * tpw, tpw), :]
        out_ref[j] = lax.dot_general(
            w, xj, (((1,), (1,)), ((), ())),
            preferred_element_type=jnp.float32)


def _sc_gate_body(l_hbm, vals_hbm, idx_hbm, tile, vout, iout):
    tpw = tile.shape[1]
    wid = lax.axis_index("s") * _NC + lax.axis_index("c")
    base = wid * tpw
    pltpu.sync_copy(l_hbm.at[wid], tile)
    for g in range(tpw // _L):
        sl = pl.ds(g * _L, _L)
        ls = [tile[e, sl] for e in range(NUM_EXPERTS)]
        m = ls[0]
        for e in range(1, NUM_EXPERTS):
            m = jnp.maximum(m, ls[e])
        s = jnp.exp(ls[0] - m)
        for e in range(1, NUM_EXPERTS):
            s = s + jnp.exp(ls[e] - m)
        idx1 = jnp.full((_L,), NUM_EXPERTS, jnp.int32)
        for e in range(NUM_EXPERTS):
            ev = jnp.full((_L,), e, jnp.int32)
            idx1 = jnp.minimum(idx1, jnp.where(ls[e] == m, ev, NUM_EXPERTS))
        m2 = jnp.full((_L,), -jnp.inf, jnp.float32)
        for e in range(NUM_EXPERTS):
            ev = jnp.full((_L,), e, jnp.int32)
            m2 = jnp.maximum(m2, jnp.where(ev == idx1, -jnp.inf, ls[e]))
        idx2 = jnp.full((_L,), NUM_EXPERTS, jnp.int32)
        for e in range(NUM_EXPERTS):
            ev = jnp.full((_L,), e, jnp.int32)
            hit = jnp.logical_and(ls[e] == m2, ev != idx1)
            idx2 = jnp.minimum(idx2, jnp.where(hit, ev, NUM_EXPERTS))
        val1 = 1.0 / s
        val2 = jnp.exp(m2 - m) / s
        vout[0, sl] = val1
        vout[1, sl] = val2
        iout[0, sl] = idx1
        iout[1, sl] = idx2
    pltpu.sync_copy(vout, vals_hbm.at[wid])
    pltpu.sync_copy(iout, idx_hbm.at[wid])


@jax.jit
def kernel(hidden_states, weight):
    x = hidden_states.reshape(-1, hidden_states.shape[-1])
    t, d = x.shape
    tpw = t // NW
    logits = pl.pallas_call(
        _matmul_body,
        grid=(t // BLOCK_T,),
        in_specs=[
            pl.BlockSpec((BLOCK_T, d), lambda i: (i, 0)),
            pl.BlockSpec((NUM_EXPERTS, d), lambda i: (0, 0)),
        ],
        out_specs=pl.BlockSpec(
            (BLOCK_T // tpw, NUM_EXPERTS, tpw), lambda i: (i, 0, 0)),
        out_shape=jax.ShapeDtypeStruct((NW, NUM_EXPERTS, tpw), jnp.float32),
    )(x, weight)

    sc_gate = functools.partial(
        pl.kernel,
        mesh=plsc.VectorSubcoreMesh(core_axis_name="c", subcore_axis_name="s"),
        out_type=[
            jax.ShapeDtypeStruct((NW, TOP_K, tpw), jnp.float32),
            jax.ShapeDtypeStruct((NW, TOP_K, tpw), jnp.int32),
        ],
        scratch_types=[
            pltpu.VMEM((NUM_EXPERTS, tpw), jnp.float32),
            pltpu.VMEM((TOP_K, tpw), jnp.float32),
            pltpu.VMEM((TOP_K, tpw), jnp.int32),
        ],
    )(_sc_gate_body)
    vals_w, idx_w = sc_gate(logits)
    vals = vals_w.transpose(0, 2, 1).reshape(t, TOP_K)
    idx = idx_w.transpose(0, 2, 1).reshape(t, TOP_K)
    return vals, idx
